# Initial kernel scaffold; baseline (speedup 1.0000x reference)
import jax
import jax.numpy as jnp
from jax.experimental import pallas as pl


def kernel(hidden_states, gate_W, Wup, bup, Wdown, bdown):
    raise NotImplementedError("write your pallas kernel here")


# speedup vs baseline: 2.6152x; 2.6152x over previous
import functools

import jax
import jax.numpy as jnp
from jax import lax
from jax.experimental import pallas as pl
from jax.experimental.pallas import tpu as pltpu


def _erf(x):
    a1, a2, a3, a4, a5 = (0.254829592, -0.284496736, 1.421413741,
                          -1.453152027, 1.061405429)
    p = 0.3275911
    s = jnp.sign(x)
    z = jnp.abs(x)
    t = 1.0 / (1.0 + p * z)
    poly = t * (a1 + t * (a2 + t * (a3 + t * (a4 + t * a5))))
    y = 1.0 - poly * jnp.exp(-z * z)
    return s * y


def _gelu(x):
    return 0.5 * x * (1.0 + _erf(x * 0.7071067811865476))


def _router_body(x_ref, gw_ref, coef_ref):
    x = x_ref[...]
    gw = gw_ref[...]
    logits = lax.dot_general(x, gw, (((1,), (1,)), ((), ())),
                             preferred_element_type=jnp.float32)
    m = jnp.max(logits, axis=1, keepdims=True)
    ex = jnp.exp(logits - m)
    probs = ex / jnp.sum(ex, axis=1, keepdims=True)
    n, e = probs.shape
    iota = lax.broadcasted_iota(jnp.int32, (n, e), 1)
    m1 = jnp.max(probs, axis=1, keepdims=True)
    a1 = jnp.min(jnp.where(probs == m1, iota, e), axis=1, keepdims=True)
    probs2 = jnp.where(iota == a1, -1.0, probs)
    m2 = jnp.max(probs2, axis=1, keepdims=True)
    a2 = jnp.min(jnp.where(probs2 == m2, iota, e), axis=1, keepdims=True)
    coef = (m1 * (iota == a1).astype(jnp.float32)
            + m2 * (iota == a2).astype(jnp.float32))
    coef_ref[...] = coef


def _ffn_body(x_ref, wup_ref, bup_ref, wdown_ref, bdown_ref, coef_ref,
              out_ref, acc_ref, *, nf):
    e = pl.program_id(1)
    f = pl.program_id(2)

    @pl.when(f == 0)
    def _():
        acc_ref[...] = jnp.zeros_like(acc_ref)

    x = x_ref[...]
    h = lax.dot_general(x, wup_ref[0], (((1,), (0,)), ((), ())),
                        preferred_element_type=jnp.float32)
    h = _gelu(h + bup_ref[0])
    acc_ref[...] += lax.dot_general(h, wdown_ref[0], (((1,), (0,)), ((), ())),
                                    preferred_element_type=jnp.float32)

    @pl.when(f == nf - 1)
    def _():
        coef = coef_ref[...]
        ne = coef.shape[1]
        onehot = (lax.broadcasted_iota(jnp.int32, (ne, 1), 0) == e
                  ).astype(jnp.float32)
        ccol = jnp.dot(coef, onehot, preferred_element_type=jnp.float32)
        contrib = ccol * (acc_ref[...] + bdown_ref[0])

        @pl.when(e == 0)
        def _():
            out_ref[...] = contrib

        @pl.when(e != 0)
        def _():
            out_ref[...] += contrib


def kernel(hidden_states, gate_W, Wup, bup, Wdown, bdown):
    B, S, D = hidden_states.shape
    E, _, DFF = Wup.shape
    N = B * S
    x = hidden_states.reshape(N, D)

    coef = pl.pallas_call(
        _router_body,
        out_shape=jax.ShapeDtypeStruct((N, E), jnp.float32),
    )(x, gate_W)

    TB = min(512, N)
    FT = min(1024, DFF)
    nt, nf = N // TB, DFF // FT

    bup3 = bup.reshape(E, 1, DFF)
    bdown3 = bdown.reshape(E, 1, D)

    out = pl.pallas_call(
        functools.partial(_ffn_body, nf=nf),
        grid=(nt, E, nf),
        in_specs=[
            pl.BlockSpec((TB, D), lambda i, e, f: (i, 0)),
            pl.BlockSpec((1, D, FT), lambda i, e, f: (e, 0, f)),
            pl.BlockSpec((1, 1, FT), lambda i, e, f: (e, 0, f)),
            pl.BlockSpec((1, FT, D), lambda i, e, f: (e, f, 0)),
            pl.BlockSpec((1, 1, D), lambda i, e, f: (e, 0, 0)),
            pl.BlockSpec((TB, E), lambda i, e, f: (i, 0)),
        ],
        out_specs=pl.BlockSpec((TB, D), lambda i, e, f: (i, 0)),
        out_shape=jax.ShapeDtypeStruct((N, D), jnp.float32),
        scratch_shapes=[pltpu.VMEM((TB, D), jnp.float32)],
    )(x, Wup, bup3, Wdown, bdown3, coef)

    return out.reshape(B, S, D)
